# Initial kernel scaffold; baseline (speedup 1.0000x reference)
import jax
import jax.numpy as jnp
from jax.experimental import pallas as pl


def kernel(teacher_boxes, student_boxes, gt_boxes):
    raise NotImplementedError("write your pallas kernel here")


# speedup vs baseline: 270.5931x; 270.5931x over previous
import jax
import jax.numpy as jnp
from jax.experimental import pallas as pl
from jax.experimental.pallas import tpu as pltpu

_IOU_THRESHOLD = 0.6
_K = 50
_LANES = 128
_RB = 8
_NPAD = 5120
_ROWS = _NPAD // _LANES
_NBLK = _ROWS // _RB


def _clip_dt(pu, pv, du, dv, rxu, rxv, hx, hy):
    ta = -(hx + pu) * rxu
    tb = (hx - pu) * rxu
    lo_u = jnp.minimum(ta, tb)
    hi_u = jnp.maximum(ta, tb)
    tc = -(hy + pv) * rxv
    td = (hy - pv) * rxv
    lo_v = jnp.minimum(tc, td)
    hi_v = jnp.maximum(tc, td)
    t0 = jnp.maximum(jnp.maximum(lo_u, lo_v), 0.0)
    t1 = jnp.minimum(jnp.minimum(hi_u, hi_v), 1.0)
    return jnp.maximum(t1 - t0, 0.0)


def _assign_body(pred_ref, gt_ref, out_ref):
    x = pred_ref[0, 0]
    y = pred_ref[0, 1]
    z = pred_ref[0, 2]
    dxa = pred_ref[0, 3]
    dya = pred_ref[0, 4]
    dza = pred_ref[0, 5]
    r = pred_ref[0, 6]

    ca = jnp.cos(r)
    sa = jnp.sin(r)
    hxa = 0.5 * dxa
    hya = 0.5 * dya
    cxh = ca * hxa
    sxh = sa * hxa
    cyh = ca * hya
    syh = sa * hya
    pax = (x + cxh - syh, x - cxh - syh, x - cxh + syh, x + cxh + syh)
    pay = (y + sxh + cyh, y - sxh + cyh, y - sxh - cyh, y + sxh - cyh)
    e0x = -2.0 * cxh
    e0y = -2.0 * sxh
    e1x = 2.0 * syh
    e1y = -2.0 * cyh

    za1 = z - 0.5 * dza
    za2 = z + 0.5 * dza
    va = dxa * dya * dza

    shape = x.shape
    best0 = jnp.full(shape, -1.0, jnp.float32)
    bki0 = jnp.zeros(shape, jnp.int32)

    def step(k, carry):
        best, bki = carry
        cx = gt_ref[0, k, 0]
        cy = gt_ref[0, k, 1]
        cb = gt_ref[0, k, 2]
        sb = gt_ref[0, k, 3]
        hx = gt_ref[0, k, 4]
        hy = gt_ref[0, k, 5]
        zb1 = gt_ref[0, k, 6]
        zb2 = gt_ref[0, k, 7]
        vb = gt_ref[0, k, 8]
        qx0 = gt_ref[0, k, 9]
        qy0 = gt_ref[0, k, 10]
        qx1 = gt_ref[0, k, 11]
        qy1 = gt_ref[0, k, 12]
        qx2 = gt_ref[0, k, 13]
        qy2 = gt_ref[0, k, 14]
        qx3 = gt_ref[0, k, 15]
        qy3 = gt_ref[0, k, 16]
        d0x = gt_ref[0, k, 17]
        d0y = gt_ref[0, k, 18]
        d1x = gt_ref[0, k, 19]
        d1y = gt_ref[0, k, 20]

        us = []
        vs = []
        for j in range(4):
            tx = pax[j] - cx
            ty = pay[j] - cy
            us.append(cb * tx + sb * ty)
            vs.append(cb * ty - sb * tx)
        e0u = cb * e0x + sb * e0y
        e0v = cb * e0y - sb * e0x
        e1u = cb * e1x + sb * e1y
        e1v = cb * e1y - sb * e1x
        r0u = 1.0 / e0u
        r0v = 1.0 / e0v
        r1u = 1.0 / e1u
        r1v = 1.0 / e1v
        eds = ((e0u, e0v, r0u, r0v), (e1u, e1v, r1u, r1v),
               (-e0u, -e0v, -r0u, -r0v), (-e1u, -e1v, -r1u, -r1v))
        area2 = jnp.zeros(shape, jnp.float32)
        for j in range(4):
            du, dv, ru, rv = eds[j]
            dt = _clip_dt(us[j], vs[j], du, dv, ru, rv, hx, hy)
            area2 = area2 + dt * (us[j] * dv - vs[j] * du)

        qs = []
        for qxj, qyj in ((qx0, qy0), (qx1, qy1), (qx2, qy2), (qx3, qy3)):
            tx = qxj - x
            ty = qyj - y
            qs.append((ca * tx + sa * ty, ca * ty - sa * tx))
        d0u = ca * d0x + sa * d0y
        d0v = ca * d0y - sa * d0x
        d1u = ca * d1x + sa * d1y
        d1v = ca * d1y - sa * d1x
        s0u = 1.0 / d0u
        s0v = 1.0 / d0v
        s1u = 1.0 / d1u
        s1v = 1.0 / d1v
        bds = ((d0u, d0v, s0u, s0v), (d1u, d1v, s1u, s1v),
               (-d0u, -d0v, -s0u, -s0v), (-d1u, -d1v, -s1u, -s1v))
        dtsum = jnp.zeros(shape, jnp.float32)
        for j in range(4):
            du, dv, ru, rv = bds[j]
            dtsum = dtsum + _clip_dt(qs[j][0], qs[j][1], du, dv, ru, rv,
                                     hxa, hya)
        area2 = area2 + dtsum * (2.0 * hx * hy)

        area = jnp.maximum(0.5 * area2, 0.0)
        h = jnp.maximum(jnp.minimum(za2, zb2) - jnp.maximum(za1, zb1), 0.0)
        inter = area * h
        iou = inter / jnp.maximum(va + vb - inter, 1e-6)
        upd = iou > best
        best = jnp.where(upd, iou, best)
        bki = jnp.where(upd, k, bki)
        return best, bki

    best, bki = jax.lax.fori_loop(0, _K, step, (best0, bki0))
    out_ref[0] = jnp.where(best < _IOU_THRESHOLD, -1, bki)


def _gt_scalars(gt):
    cx = gt[..., 0]
    cy = gt[..., 1]
    zc = gt[..., 2]
    dx = gt[..., 3]
    dy = gt[..., 4]
    dz = gt[..., 5]
    rr = gt[..., 6]
    cb = jnp.cos(rr)
    sb = jnp.sin(rr)
    hx = 0.5 * dx
    hy = 0.5 * dy
    cxh = cb * hx
    sxh = sb * hx
    cyh = cb * hy
    syh = sb * hy
    qx0 = cx + cxh - syh
    qy0 = cy + sxh + cyh
    qx1 = cx - cxh - syh
    qy1 = cy - sxh + cyh
    qx2 = cx - cxh + syh
    qy2 = cy - sxh - cyh
    qx3 = cx + cxh + syh
    qy3 = cy + sxh - cyh
    d0x = qx1 - qx0
    d0y = qy1 - qy0
    d1x = qx2 - qx1
    d1y = qy2 - qy1
    zb1 = zc - 0.5 * dz
    zb2 = zc + 0.5 * dz
    vb = dx * dy * dz
    pad = jnp.zeros_like(cx)
    return jnp.stack([cx, cy, cb, sb, hx, hy, zb1, zb2, vb,
                      qx0, qy0, qx1, qy1, qx2, qy2, qx3, qy3,
                      d0x, d0y, d1x, d1y, pad, pad, pad], axis=-1)


def kernel(teacher_boxes, student_boxes, gt_boxes):
    B, N, _ = teacher_boxes.shape
    pred = jnp.concatenate([teacher_boxes, student_boxes], axis=0)
    padbox = jnp.zeros((2 * B, _NPAD - N, 7), jnp.float32).at[:, :, 3:6].set(1.0)
    pred = jnp.concatenate([pred, padbox], axis=1)
    planes = jnp.transpose(pred, (0, 2, 1))
    planes = jnp.concatenate(
        [planes, jnp.zeros((2 * B, 1, _NPAD), jnp.float32)], axis=1)
    planes = planes.reshape(2 * B, 8, _ROWS, _LANES)

    gtp = _gt_scalars(gt_boxes)
    gtp = jnp.tile(gtp, (2, 1, 1))

    out = pl.pallas_call(
        _assign_body,
        grid=(2 * B, _NBLK),
        in_specs=[
            pl.BlockSpec((1, 8, _RB, _LANES), lambda c, nb: (c, 0, nb, 0)),
            pl.BlockSpec((1, _K, 24), lambda c, nb: (c, 0, 0),
                         memory_space=pltpu.SMEM),
        ],
        out_specs=pl.BlockSpec((1, _RB, _LANES), lambda c, nb: (c, nb, 0)),
        out_shape=jax.ShapeDtypeStruct((2 * B, _ROWS, _LANES), jnp.int32),
    )(planes, gtp)

    out = out.reshape(2 * B, _NPAD)[:, :N]
    return out[:B], out[B:]
